# baseline (speedup 1.0000x reference)
import functools

import jax
import jax.numpy as jnp
from jax import lax
from jax.experimental import pallas as pl
from jax.experimental.pallas import tpu as pltpu
from jax.experimental.pallas import tpu_sc as plsc

_B, _L, _D = 4, 4096, 1024
_NC, _NS = 2, 16
_NW = _NC * _NS
_LPW = _L // _NW
_CH = 32
_NCH = _LPW // _CH
_CHW = _CH * _D
_NV = _CHW // 16

_mesh = plsc.VectorSubcoreMesh(
    core_axis_name="c", subcore_axis_name="s",
    num_cores=_NC, num_subcores=_NS,
)


@functools.partial(
    pl.kernel,
    out_type=jax.ShapeDtypeStruct((_B * _L * _D,), jnp.float32),
    mesh=_mesh,
    scratch_types=[
        pltpu.VMEM((_CHW,), jnp.float32),
        pltpu.VMEM((_CHW,), jnp.float32),
    ],
)
def _pos_add(feat_hbm, pos_hbm, out_hbm, pos_v, feat_v):
    wid = lax.axis_index("s") * _NC + lax.axis_index("c")
    base = wid * (_LPW * _D)

    def chunk(c, carry):
        off = base + c * _CHW
        pltpu.sync_copy(pos_hbm.at[pl.ds(off, _CHW)], pos_v)
        for b in range(_B):
            foff = b * (_L * _D) + off
            pltpu.sync_copy(feat_hbm.at[pl.ds(foff, _CHW)], feat_v)

            @plsc.parallel_loop(0, _NV, 1, unroll=8)
            def _add(i):
                s = i * 16
                feat_v[pl.ds(s, 16)] = feat_v[pl.ds(s, 16)] + pos_v[pl.ds(s, 16)]

            pltpu.sync_copy(feat_v, out_hbm.at[pl.ds(foff, _CHW)])
        return carry

    lax.fori_loop(0, _NCH, chunk, 0)


def kernel(feat, pos_emb_table):
    flat_feat = feat.reshape(_B * _L * _D)
    flat_pos = pos_emb_table.reshape(_L * _D)
    out = _pos_add(flat_feat, flat_pos)
    return out.reshape(_B, _L, _D)


# speedup vs baseline: 1.1921x; 1.1921x over previous
import functools

import jax
import jax.numpy as jnp
from jax import lax
from jax.experimental import pallas as pl
from jax.experimental.pallas import tpu as pltpu
from jax.experimental.pallas import tpu_sc as plsc

_B, _L, _D = 4, 4096, 1024
_NC, _NS = 2, 16
_NW = _NC * _NS
_LPW = _L // _NW
_CH = 8
_NCH = _LPW // _CH
_CHW = _CH * _D
_NV = _CHW // 16
_SLOTS = 8
_LOOKAHEAD = 4
_UNITS = _NCH * _B

_mesh = plsc.VectorSubcoreMesh(
    core_axis_name="c", subcore_axis_name="s",
    num_cores=_NC, num_subcores=_NS,
)


@functools.partial(
    pl.kernel,
    out_type=jax.ShapeDtypeStruct((_B * _L * _D,), jnp.float32),
    mesh=_mesh,
    scratch_types=[
        [pltpu.VMEM((_CHW,), jnp.float32) for _ in range(2)],
        [pltpu.VMEM((_CHW,), jnp.float32) for _ in range(_SLOTS)],
        [pltpu.SemaphoreType.DMA for _ in range(2)],
        [pltpu.SemaphoreType.DMA for _ in range(_SLOTS)],
        [pltpu.SemaphoreType.DMA for _ in range(_SLOTS)],
    ],
)
def _pos_add(feat_hbm, pos_hbm, out_hbm, pos_v, feat_v, pos_sem, in_sem, out_sem):
    wid = lax.axis_index("s") * _NC + lax.axis_index("c")
    base = wid * (_LPW * _D)

    def in_copy(u):
        c, b = divmod(u, _B)
        s = u % _SLOTS
        off = b * (_L * _D) + base + c * _CHW
        return pltpu.async_copy(
            feat_hbm.at[pl.ds(off, _CHW)], feat_v[s], in_sem[s])

    def out_copy(u):
        c, b = divmod(u, _B)
        s = u % _SLOTS
        off = b * (_L * _D) + base + c * _CHW
        return pltpu.async_copy(
            feat_v[s], out_hbm.at[pl.ds(off, _CHW)], out_sem[s])

    def pos_copy(c):
        return pltpu.async_copy(
            pos_hbm.at[pl.ds(base + c * _CHW, _CHW)], pos_v[c % 2],
            pos_sem[c % 2])

    in_h = [None] * _SLOTS
    out_h = [None] * _SLOTS
    pos_h = [None, None]

    pos_h[0] = pos_copy(0)
    pos_h[1] = pos_copy(1)
    for u in range(_LOOKAHEAD):
        in_h[u % _SLOTS] = in_copy(u)

    for u in range(_UNITS):
        c, b = divmod(u, _B)
        s = u % _SLOTS
        if b == 0:
            pos_h[c % 2].wait()
            if c + 1 < _NCH and c >= 1:
                pos_h[(c + 1) % 2] = pos_copy(c + 1)
        in_h[s].wait()
        fv = feat_v[s]
        pv = pos_v[c % 2]

        @plsc.parallel_loop(0, _NV, 1, unroll=8)
        def _add(i):
            t = i * 16
            fv[pl.ds(t, 16)] = fv[pl.ds(t, 16)] + pv[pl.ds(t, 16)]

        out_h[s] = out_copy(u)
        nu = u + _LOOKAHEAD
        if nu < _UNITS:
            ns = nu % _SLOTS
            if out_h[ns] is not None:
                out_h[ns].wait()
            in_h[ns] = in_copy(nu)

    for s in range(_SLOTS):
        if out_h[s] is not None:
            out_h[s].wait()


def kernel(feat, pos_emb_table):
    flat_feat = feat.reshape(_B * _L * _D)
    flat_pos = pos_emb_table.reshape(_L * _D)
    out = _pos_add(flat_feat, flat_pos)
    return out.reshape(_B, _L, _D)


# speedup vs baseline: 3.2660x; 2.7397x over previous
import functools

import jax
import jax.numpy as jnp
from jax import lax
from jax.experimental import pallas as pl
from jax.experimental.pallas import tpu as pltpu
from jax.experimental.pallas import tpu_sc as plsc

_B, _L, _D = 4, 4096, 1024
_NC, _NS = 2, 16
_NW = _NC * _NS
_LPW = _L // _NW
_CH = 8
_NCH = _LPW // _CH
_NVR = _D // 16
_SLOTS = 10
_LOOKAHEAD = 6
_UNITS = _NCH * _B

_mesh = plsc.VectorSubcoreMesh(
    core_axis_name="c", subcore_axis_name="s",
    num_cores=_NC, num_subcores=_NS,
)


@functools.partial(
    pl.kernel,
    out_type=jax.ShapeDtypeStruct((_B, _L, _D), jnp.float32),
    mesh=_mesh,
    compiler_params=pltpu.CompilerParams(use_tc_tiling_on_sc=True),
    scratch_types=[
        [pltpu.VMEM((_CH, _D), jnp.float32) for _ in range(2)],
        [pltpu.VMEM((_CH, _D), jnp.float32) for _ in range(_SLOTS)],
        [pltpu.SemaphoreType.DMA for _ in range(2)],
        [pltpu.SemaphoreType.DMA for _ in range(_SLOTS)],
        [pltpu.SemaphoreType.DMA for _ in range(_SLOTS)],
    ],
)
def _pos_add(feat_hbm, pos_hbm, out_hbm, pos_v, feat_v, pos_sem, in_sem, out_sem):
    wid = lax.axis_index("s") * _NC + lax.axis_index("c")
    base = wid * _LPW

    def in_copy(u):
        c, b = divmod(u, _B)
        s = u % _SLOTS
        return pltpu.async_copy(
            feat_hbm.at[b, pl.ds(base + c * _CH, _CH), :], feat_v[s], in_sem[s])

    def out_copy(u):
        c, b = divmod(u, _B)
        s = u % _SLOTS
        return pltpu.async_copy(
            feat_v[s], out_hbm.at[b, pl.ds(base + c * _CH, _CH), :], out_sem[s])

    def pos_copy(c):
        return pltpu.async_copy(
            pos_hbm.at[pl.ds(base + c * _CH, _CH), :], pos_v[c % 2],
            pos_sem[c % 2])

    in_h = [None] * _SLOTS
    out_h = [None] * _SLOTS
    pos_h = [None, None]

    pos_h[0] = pos_copy(0)
    pos_h[1] = pos_copy(1)
    for u in range(_LOOKAHEAD):
        in_h[u % _SLOTS] = in_copy(u)

    for u in range(_UNITS):
        c, b = divmod(u, _B)
        s = u % _SLOTS
        if b == 0:
            pos_h[c % 2].wait()
            if c + 1 < _NCH and c >= 1:
                pos_h[(c + 1) % 2] = pos_copy(c + 1)
        in_h[s].wait()
        fv = feat_v[s]
        pv = pos_v[c % 2]

        @plsc.parallel_loop(0, _CH * _NVR, 1, unroll=8)
        def _add(i):
            r = i >> 6
            t = (i & (_NVR - 1)) * 16
            fv[r, pl.ds(t, 16)] = fv[r, pl.ds(t, 16)] + pv[r, pl.ds(t, 16)]

        out_h[s] = out_copy(u)
        nu = u + _LOOKAHEAD
        if nu < _UNITS:
            ns = nu % _SLOTS
            if out_h[ns] is not None:
                out_h[ns].wait()
            in_h[ns] = in_copy(nu)

    for s in range(_SLOTS):
        if out_h[s] is not None:
            out_h[s].wait()


def kernel(feat, pos_emb_table):
    return _pos_add(feat, pos_emb_table)


# speedup vs baseline: 3.3469x; 1.0248x over previous
import functools

import jax
import jax.numpy as jnp
from jax import lax
from jax.experimental import pallas as pl
from jax.experimental.pallas import tpu as pltpu
from jax.experimental.pallas import tpu_sc as plsc

_B, _L, _D = 4, 4096, 1024
_NC, _NS = 2, 16
_NW = _NC * _NS
_LPW = _L // _NW
_CH = 16
_NCH = _LPW // _CH
_NVR = _D // 16
_SLOTS = 5
_LOOKAHEAD = 3
_UNITS = _NCH * _B

_mesh = plsc.VectorSubcoreMesh(
    core_axis_name="c", subcore_axis_name="s",
    num_cores=_NC, num_subcores=_NS,
)


@functools.partial(
    pl.kernel,
    out_type=jax.ShapeDtypeStruct((_B, _L, _D), jnp.float32),
    mesh=_mesh,
    compiler_params=pltpu.CompilerParams(use_tc_tiling_on_sc=True),
    scratch_types=[
        [pltpu.VMEM((_CH, _D), jnp.float32) for _ in range(2)],
        [pltpu.VMEM((_CH, _D), jnp.float32) for _ in range(_SLOTS)],
        [pltpu.SemaphoreType.DMA for _ in range(2)],
        [pltpu.SemaphoreType.DMA for _ in range(_SLOTS)],
        [pltpu.SemaphoreType.DMA for _ in range(_SLOTS)],
    ],
)
def _pos_add(feat_hbm, pos_hbm, out_hbm, pos_v, feat_v, pos_sem, in_sem, out_sem):
    wid = lax.axis_index("s") * _NC + lax.axis_index("c")
    base = wid * _LPW

    def in_copy(u):
        c, b = divmod(u, _B)
        s = u % _SLOTS
        return pltpu.async_copy(
            feat_hbm.at[b, pl.ds(base + c * _CH, _CH), :], feat_v[s], in_sem[s])

    def out_copy(u):
        c, b = divmod(u, _B)
        s = u % _SLOTS
        return pltpu.async_copy(
            feat_v[s], out_hbm.at[b, pl.ds(base + c * _CH, _CH), :], out_sem[s])

    def pos_copy(c):
        return pltpu.async_copy(
            pos_hbm.at[pl.ds(base + c * _CH, _CH), :], pos_v[c % 2],
            pos_sem[c % 2])

    in_h = [None] * _SLOTS
    out_h = [None] * _SLOTS
    pos_h = [None, None]

    pos_h[0] = pos_copy(0)
    pos_h[1] = pos_copy(1)
    for u in range(_LOOKAHEAD):
        in_h[u % _SLOTS] = in_copy(u)

    for u in range(_UNITS):
        c, b = divmod(u, _B)
        s = u % _SLOTS
        if b == 0:
            pos_h[c % 2].wait()
            if c + 1 < _NCH and c >= 1:
                pos_h[(c + 1) % 2] = pos_copy(c + 1)
        in_h[s].wait()
        fv = feat_v[s]
        pv = pos_v[c % 2]

        @plsc.parallel_loop(0, _CH * _NVR, 1, unroll=8)
        def _add(i):
            r = i >> 6
            t = (i & (_NVR - 1)) * 16
            fv[r, pl.ds(t, 16)] = fv[r, pl.ds(t, 16)] + pv[r, pl.ds(t, 16)]

        out_h[s] = out_copy(u)
        nu = u + _LOOKAHEAD
        if nu < _UNITS:
            ns = nu % _SLOTS
            if out_h[ns] is not None:
                out_h[ns].wait()
            in_h[ns] = in_copy(nu)

    for s in range(_SLOTS):
        if out_h[s] is not None:
            out_h[s].wait()


def kernel(feat, pos_emb_table):
    return _pos_add(feat, pos_emb_table)


# speedup vs baseline: 3.5336x; 1.0558x over previous
import functools

import jax
import jax.numpy as jnp
from jax import lax
from jax.experimental import pallas as pl
from jax.experimental.pallas import tpu as pltpu
from jax.experimental.pallas import tpu_sc as plsc

_B, _L, _D = 4, 4096, 1024
_NC, _NS = 2, 16
_NW = _NC * _NS
_LPW = _L // _NW
_CH = 8
_NCH = _LPW // _CH
_NVR = _D // 16

_mesh = plsc.VectorSubcoreMesh(
    core_axis_name="c", subcore_axis_name="s",
    num_cores=_NC, num_subcores=_NS,
)


@functools.partial(
    pl.kernel,
    out_type=jax.ShapeDtypeStruct((_B, _L, _D), jnp.float32),
    mesh=_mesh,
    compiler_params=pltpu.CompilerParams(use_tc_tiling_on_sc=True),
    scratch_types=[
        [pltpu.VMEM((_CH, _D), jnp.float32) for _ in range(2)],
        [pltpu.VMEM((_CH, _D), jnp.float32) for _ in range(_B)],
        [pltpu.VMEM((_CH, _D), jnp.float32) for _ in range(_B)],
        [pltpu.SemaphoreType.DMA for _ in range(2)],
        [pltpu.SemaphoreType.DMA for _ in range(_B)],
        [pltpu.SemaphoreType.DMA for _ in range(_B)],
    ],
)
def _pos_add(feat_hbm, pos_hbm, out_hbm, pos_v, fin_v, fout_v,
             pos_sem, in_sem, out_sem):
    wid = lax.axis_index("s") * _NC + lax.axis_index("c")
    base = wid * _LPW

    def row0(c):
        return base + c * _CH

    def issue_in(c, b):
        pltpu.async_copy(
            feat_hbm.at[b, pl.ds(row0(c), _CH), :], fin_v[b], in_sem[b])

    pltpu.async_copy(pos_hbm.at[pl.ds(row0(0), _CH), :], pos_v[0], pos_sem[0])
    pltpu.async_copy(pos_hbm.at[pl.ds(row0(1), _CH), :], pos_v[1], pos_sem[1])
    for b in range(_B):
        issue_in(0, b)

    def half(c2, carry):
        for k in range(2):
            c = c2 * 2 + k
            pltpu.make_async_copy(
                pos_hbm.at[pl.ds(row0(0), _CH), :], pos_v[k],
                pos_sem[k]).wait()
            for b in range(_B):
                pltpu.make_async_copy(
                    feat_hbm.at[b, pl.ds(row0(0), _CH), :], fin_v[b],
                    in_sem[b]).wait()

                @pl.when(c > 0)
                def _():
                    pltpu.make_async_copy(
                        fout_v[b], out_hbm.at[b, pl.ds(row0(0), _CH), :],
                        out_sem[b]).wait()

                fv, gv, pv = fin_v[b], fout_v[b], pos_v[k]

                @plsc.parallel_loop(0, _CH * _NVR, 1, unroll=8)
                def _add(i):
                    r = i >> 6
                    t = (i & (_NVR - 1)) * 16
                    gv[r, pl.ds(t, 16)] = fv[r, pl.ds(t, 16)] + pv[r, pl.ds(t, 16)]

                pltpu.async_copy(
                    fout_v[b], out_hbm.at[b, pl.ds(row0(c), _CH), :],
                    out_sem[b])

                @pl.when(c + 1 < _NCH)
                def _():
                    issue_in(c + 1, b)

            @pl.when(c + 2 < _NCH)
            def _():
                pltpu.async_copy(
                    pos_hbm.at[pl.ds(row0(c + 2), _CH), :], pos_v[k],
                    pos_sem[k])
        return carry

    lax.fori_loop(0, _NCH // 2, half, 0)

    for b in range(_B):
        pltpu.make_async_copy(
            fout_v[b], out_hbm.at[b, pl.ds(row0(_NCH - 1), _CH), :],
            out_sem[b]).wait()


def kernel(feat, pos_emb_table):
    return _pos_add(feat, pos_emb_table)
